# Initial kernel scaffold; baseline (speedup 1.0000x reference)
import jax
import jax.numpy as jnp
from jax.experimental import pallas as pl


def kernel(rating_matrix, x_row, x_col, x_val):
    raise NotImplementedError("write your pallas kernel here")


# speedup vs baseline: 9.9971x; 9.9971x over previous
import functools

import numpy as np
import jax
import jax.numpy as jnp
from jax import lax
from jax.experimental import pallas as pl
from jax.experimental.pallas import tpu as pltpu
from jax.experimental.pallas import tpu_sc as plsc

M = 20000
MP = 20096
N = 16384
B = 32
DEGREE = 6
ALPHA = 500.0
BETA = 0.5
LAM_MAX = 100.0
LAM_MIN = 0.0

NUM_CORES = 2
NUM_SUBCORES = 16
NUM_TILES = NUM_CORES * NUM_SUBCORES
K = 128
CHB = 16
EDGES_PER_CHUNK = K * CHB


def _cheby_coeffs():
    exponent = 1.0 / (1.0 + BETA)
    j = np.arange(DEGREE + 1)
    theta = np.pi * (j + 0.5) / (DEGREE + 1)
    mid, half = (LAM_MAX + LAM_MIN) / 2.0, (LAM_MAX - LAM_MIN) / 2.0
    lam_nodes = mid + half * np.cos(theta)
    lam_pow = np.power(np.maximum(lam_nodes, 1e-12), exponent)
    f_nodes = lam_pow / (lam_pow + ALPHA)
    coeffs = np.zeros(DEGREE + 1)
    for k in range(DEGREE + 1):
        coeffs[k] = 2.0 / (DEGREE + 1) * np.sum(f_nodes * np.cos(k * theta))
    coeffs[0] /= 2.0
    return [float(c) for c in coeffs], float(mid), float(half)


_COEFFS, _MID, _HALF = _cheby_coeffs()


_BCAST_DN = lax.GatherDimensionNumbers(
    offset_dims=(), collapsed_slice_dims=(0,), start_index_map=(0,))


def _lane_bcast(v16, lane):
    return lax.gather(v16, jnp.full((16, 1), lane, jnp.int32), _BCAST_DN, (1,),
                      mode=lax.GatherScatterMode.PROMISE_IN_BOUNDS)


@functools.lru_cache(maxsize=None)
def _make_spmm(src_dim, dst_dim, nnz_pad):
    per_tile = nnz_pad // NUM_TILES
    nchunk = per_tile // EDGES_PER_CHUNK
    rpt = dst_dim // NUM_SUBCORES
    mesh = plsc.VectorSubcoreMesh(core_axis_name="c", subcore_axis_name="s")

    @functools.partial(
        pl.kernel,
        out_type=jax.ShapeDtypeStruct((NUM_CORES, dst_dim, B), jnp.float32),
        mesh=mesh,
        compiler_params=pltpu.CompilerParams(use_tc_tiling_on_sc=False),
        scratch_types=[
            pltpu.VMEM((EDGES_PER_CHUNK,), jnp.int32),
            pltpu.VMEM((CHB, K), jnp.int32),
            pltpu.VMEM((EDGES_PER_CHUNK,), jnp.float32),
            pltpu.VMEM((K, B), jnp.float32),
            pltpu.VMEM((rpt, B), jnp.float32),
            pltpu.VMEM_SHARED((dst_dim, B), jnp.float32),
        ],
    )
    def spmm(src_hbm, gidx_hbm, sidx_hbm, val_hbm, out_hbm,
             gidx_v, sidx_v, val_v, rows_v, zbuf, acc):
        cid = lax.axis_index("c")
        sid = lax.axis_index("s")
        wid = cid * NUM_SUBCORES + sid

        zero16 = jnp.zeros((16,), jnp.float32)

        def zrow(i, carry):
            zbuf[i, pl.ds(0, 16)] = zero16
            zbuf[i, pl.ds(16, 16)] = zero16
            return carry

        lax.fori_loop(0, rpt, zrow, 0)
        pltpu.sync_copy(zbuf, acc.at[pl.ds(sid * rpt, rpt)])
        plsc.subcore_barrier()

        tile_edge_base = wid * per_tile
        tile_srow_base = wid * (per_tile // K)

        def chunk_body(ci, carry):
            ebase = tile_edge_base + ci * EDGES_PER_CHUNK
            rbase = tile_srow_base + ci * CHB
            pltpu.sync_copy(gidx_hbm.at[pl.ds(ebase, EDGES_PER_CHUNK)], gidx_v)
            pltpu.sync_copy(sidx_hbm.at[pl.ds(rbase, CHB)], sidx_v)
            pltpu.sync_copy(val_hbm.at[pl.ds(ebase, EDGES_PER_CHUNK)], val_v)

            def batch_body(j, c2):
                pltpu.sync_copy(src_hbm.at[gidx_v.at[pl.ds(j * K, K)]], rows_v)
                for g in range(K // 16):
                    val16 = val_v[pl.ds(j * K + g * 16, 16)]
                    for kk in range(16):
                        e = g * 16 + kk
                        bv = _lane_bcast(val16, kk)
                        rows_v[e, pl.ds(0, 16)] = rows_v[e, pl.ds(0, 16)] * bv
                        rows_v[e, pl.ds(16, 16)] = rows_v[e, pl.ds(16, 16)] * bv
                pltpu.sync_copy(rows_v, acc.at[sidx_v.at[j]], add=True)
                return c2

            lax.fori_loop(0, CHB, batch_body, 0)
            return carry

        lax.fori_loop(0, nchunk, chunk_body, 0)
        plsc.subcore_barrier()
        pltpu.sync_copy(acc.at[pl.ds(sid * rpt, rpt)],
                        out_hbm.at[cid, pl.ds(sid * rpt, rpt)])

    return spmm


def _add2_kernel(p_ref, o_ref):
    o_ref[...] = p_ref[0] + p_ref[1]


def _combine_partials(p):
    r = p.shape[1]
    return pl.pallas_call(
        _add2_kernel,
        out_shape=jax.ShapeDtypeStruct((r, 128), jnp.float32),
    )(p)


def _init_kernel(p_ref, v_ref, t1_ref, out_ref):
    a = p_ref[0] + p_ref[1]
    t1 = a * (1.0 / _HALF) - (_MID / _HALF) * v_ref[...]
    t1_ref[...] = t1
    out_ref[...] = _COEFFS[0] * v_ref[...] + _COEFFS[1] * t1


def _make_step_kernel(ck):
    def _step_kernel(p_ref, t1_ref, t0_ref, oin_ref, tnew_ref, onew_ref):
        a = p_ref[0] + p_ref[1]
        tnew = (2.0 / _HALF) * a - (2.0 * _MID / _HALF) * t1_ref[...] - t0_ref[...]
        tnew_ref[...] = tnew
        onew_ref[...] = oin_ref[...] + ck * tnew
    return _step_kernel


def kernel(rating_matrix, x_row, x_col, x_val):
    nnz = x_row.shape[0]
    nnz_pad = -(-nnz // (NUM_TILES * EDGES_PER_CHUNK)) * (NUM_TILES * EDGES_PER_CHUNK)
    pad = nnz_pad - nnz

    row = jnp.concatenate([x_row.astype(jnp.int32), jnp.zeros((pad,), jnp.int32)])
    col = jnp.concatenate([x_col.astype(jnp.int32), jnp.zeros((pad,), jnp.int32)])
    val = jnp.concatenate([x_val.astype(jnp.float32), jnp.zeros((pad,), jnp.float32)])
    row2d = row.reshape(-1, K)
    col2d = col.reshape(-1, K)

    spmm_x = _make_spmm(N, MP, nnz_pad)
    spmm_xt = _make_spmm(MP, N, nnz_pad)

    rn = N * B // 128
    rm = MP * B // 128

    def a_op(vv2d):
        vv = vv2d.reshape(N, B)
        up = spmm_x(vv, col, row2d, val)
        u = _combine_partials(up.reshape(NUM_CORES, rm, 128)).reshape(MP, B)
        ap = spmm_xt(u, row, col2d, val)
        return ap.reshape(NUM_CORES, rn, 128)

    v2d = rating_matrix.T.reshape(rn, 128)

    t1, out = pl.pallas_call(
        _init_kernel,
        out_shape=(jax.ShapeDtypeStruct((rn, 128), jnp.float32),
                   jax.ShapeDtypeStruct((rn, 128), jnp.float32)),
    )(a_op(v2d), v2d)

    t0 = v2d
    for k in range(2, DEGREE + 1):
        t_new, out = pl.pallas_call(
            _make_step_kernel(_COEFFS[k]),
            out_shape=(jax.ShapeDtypeStruct((rn, 128), jnp.float32),
                       jax.ShapeDtypeStruct((rn, 128), jnp.float32)),
        )(a_op(t1), t1, t0, out)
        t0, t1 = t1, t_new

    return out.reshape(N, B).T


# speedup vs baseline: 12.8439x; 1.2848x over previous
import functools

import numpy as np
import jax
import jax.numpy as jnp
from jax import lax
from jax.experimental import pallas as pl
from jax.experimental.pallas import tpu as pltpu
from jax.experimental.pallas import tpu_sc as plsc

M = 20000
MP = 20096
N = 16384
B = 32
DEGREE = 6
ALPHA = 500.0
BETA = 0.5
LAM_MAX = 100.0
LAM_MIN = 0.0

NUM_CORES = 2
NUM_SUBCORES = 16
NUM_TILES = NUM_CORES * NUM_SUBCORES
K = 128
CHB = 16
EDGES_PER_CHUNK = K * CHB


def _cheby_coeffs():
    exponent = 1.0 / (1.0 + BETA)
    j = np.arange(DEGREE + 1)
    theta = np.pi * (j + 0.5) / (DEGREE + 1)
    mid, half = (LAM_MAX + LAM_MIN) / 2.0, (LAM_MAX - LAM_MIN) / 2.0
    lam_nodes = mid + half * np.cos(theta)
    lam_pow = np.power(np.maximum(lam_nodes, 1e-12), exponent)
    f_nodes = lam_pow / (lam_pow + ALPHA)
    coeffs = np.zeros(DEGREE + 1)
    for k in range(DEGREE + 1):
        coeffs[k] = 2.0 / (DEGREE + 1) * np.sum(f_nodes * np.cos(k * theta))
    coeffs[0] /= 2.0
    return [float(c) for c in coeffs], float(mid), float(half)


_COEFFS, _MID, _HALF = _cheby_coeffs()


_BCAST_DN = lax.GatherDimensionNumbers(
    offset_dims=(), collapsed_slice_dims=(0,), start_index_map=(0,))


def _lane_bcast(v16, lane):
    return lax.gather(v16, jnp.full((16, 1), lane, jnp.int32), _BCAST_DN, (1,),
                      mode=lax.GatherScatterMode.PROMISE_IN_BOUNDS)


@functools.lru_cache(maxsize=None)
def _make_spmm(src_dim, dst_dim, nnz_pad):
    per_tile = nnz_pad // NUM_TILES
    nbatch = per_tile // K
    rpt = dst_dim // NUM_SUBCORES
    mesh = plsc.VectorSubcoreMesh(core_axis_name="c", subcore_axis_name="s")

    @functools.partial(
        pl.kernel,
        out_type=jax.ShapeDtypeStruct((NUM_CORES, dst_dim, B), jnp.float32),
        mesh=mesh,
        compiler_params=pltpu.CompilerParams(use_tc_tiling_on_sc=False),
        scratch_types=[
            pltpu.VMEM((per_tile,), jnp.int32),
            pltpu.VMEM((nbatch, K), jnp.int32),
            pltpu.VMEM((per_tile,), jnp.float32),
            pltpu.VMEM((K, B), jnp.float32),
            pltpu.VMEM((K, B), jnp.float32),
            pltpu.VMEM_SHARED((dst_dim, B), jnp.float32),
            pltpu.SemaphoreType.DMA,
            pltpu.SemaphoreType.DMA,
            pltpu.SemaphoreType.DMA,
            pltpu.SemaphoreType.DMA,
            pltpu.SemaphoreType.DMA,
        ],
    )
    def spmm(src_hbm, gidx_hbm, sidx_hbm, val_hbm, out_hbm,
             gidx_v, sidx_v, val_v, rows_a, rows_b, acc,
             gsa, gsb, ssa, ssb, stg):
        cid = lax.axis_index("c")
        sid = lax.axis_index("s")
        wid = cid * NUM_SUBCORES + sid

        ebase = wid * per_tile
        rbase = wid * nbatch
        pltpu.async_copy(gidx_hbm.at[pl.ds(ebase, per_tile)], gidx_v, stg)
        pltpu.async_copy(sidx_hbm.at[pl.ds(rbase, nbatch)], sidx_v, stg)
        pltpu.async_copy(val_hbm.at[pl.ds(ebase, per_tile)], val_v, stg)

        zero16 = jnp.zeros((16,), jnp.float32)

        def zrow(i, carry):
            rows_a[i, pl.ds(0, 16)] = zero16
            rows_a[i, pl.ds(16, 16)] = zero16
            return carry

        lax.fori_loop(0, K, zrow, 0)
        full, rem = divmod(rpt, K)
        for z in range(full):
            pltpu.sync_copy(rows_a, acc.at[pl.ds(sid * rpt + z * K, K)])
        if rem:
            pltpu.sync_copy(rows_a.at[pl.ds(0, rem)],
                            acc.at[pl.ds(sid * rpt + full * K, rem)])
        pltpu.make_async_copy(gidx_hbm.at[pl.ds(ebase, per_tile)], gidx_v, stg).wait()
        pltpu.make_async_copy(sidx_hbm.at[pl.ds(rbase, nbatch)], sidx_v, stg).wait()
        pltpu.make_async_copy(val_hbm.at[pl.ds(ebase, per_tile)], val_v, stg).wait()
        plsc.subcore_barrier()

        def issue_gather(i, buf, sem):
            pltpu.async_copy(src_hbm.at[gidx_v.at[pl.ds(i * K, K)]], buf, sem)

        def issue_scatter(i, buf, sem):
            pltpu.async_copy(buf, acc.at[sidx_v.at[i]], sem, add=True)

        def wait_rows_sem(buf, sem):
            pltpu.make_async_copy(src_hbm.at[pl.ds(0, K)], buf, sem).wait()

        def scale(buf, i):
            for g in range(K // 16):
                val16 = val_v[pl.ds(i * K + g * 16, 16)]
                for kk in range(16):
                    e = g * 16 + kk
                    bv = _lane_bcast(val16, kk)
                    buf[e, pl.ds(0, 16)] = buf[e, pl.ds(0, 16)] * bv
                    buf[e, pl.ds(16, 16)] = buf[e, pl.ds(16, 16)] * bv

        issue_gather(0, rows_a, gsa)
        wait_rows_sem(rows_a, gsa)
        issue_gather(1, rows_b, gsb)
        scale(rows_a, 0)
        issue_scatter(0, rows_a, ssa)

        def body(i, carry):
            @pl.when(i % 2 == 1)
            def _():
                wait_rows_sem(rows_b, gsb)
                wait_rows_sem(rows_a, ssa)
                @pl.when(i + 1 < nbatch)
                def _():
                    issue_gather(i + 1, rows_a, gsa)
                scale(rows_b, i)
                issue_scatter(i, rows_b, ssb)

            @pl.when(i % 2 == 0)
            def _():
                wait_rows_sem(rows_a, gsa)
                wait_rows_sem(rows_b, ssb)
                @pl.when(i + 1 < nbatch)
                def _():
                    issue_gather(i + 1, rows_b, gsb)
                scale(rows_a, i)
                issue_scatter(i, rows_a, ssa)

            return carry

        lax.fori_loop(1, nbatch, body, 0)
        wait_rows_sem(rows_b, ssb)
        plsc.subcore_barrier()
        pltpu.sync_copy(acc.at[pl.ds(sid * rpt, rpt)],
                        out_hbm.at[cid, pl.ds(sid * rpt, rpt)])

    return spmm


def _add2_kernel(p_ref, o_ref):
    o_ref[...] = p_ref[0] + p_ref[1]


def _combine_partials(p):
    r = p.shape[1]
    return pl.pallas_call(
        _add2_kernel,
        out_shape=jax.ShapeDtypeStruct((r, 128), jnp.float32),
    )(p)


def _init_kernel(p_ref, v_ref, t1_ref, out_ref):
    a = p_ref[0] + p_ref[1]
    t1 = a * (1.0 / _HALF) - (_MID / _HALF) * v_ref[...]
    t1_ref[...] = t1
    out_ref[...] = _COEFFS[0] * v_ref[...] + _COEFFS[1] * t1


def _make_step_kernel(ck):
    def _step_kernel(p_ref, t1_ref, t0_ref, oin_ref, tnew_ref, onew_ref):
        a = p_ref[0] + p_ref[1]
        tnew = (2.0 / _HALF) * a - (2.0 * _MID / _HALF) * t1_ref[...] - t0_ref[...]
        tnew_ref[...] = tnew
        onew_ref[...] = oin_ref[...] + ck * tnew
    return _step_kernel


def kernel(rating_matrix, x_row, x_col, x_val):
    nnz = x_row.shape[0]
    nnz_pad = -(-nnz // (NUM_TILES * EDGES_PER_CHUNK)) * (NUM_TILES * EDGES_PER_CHUNK)
    pad = nnz_pad - nnz

    row = jnp.concatenate([x_row.astype(jnp.int32), jnp.zeros((pad,), jnp.int32)])
    col = jnp.concatenate([x_col.astype(jnp.int32), jnp.zeros((pad,), jnp.int32)])
    val = jnp.concatenate([x_val.astype(jnp.float32), jnp.zeros((pad,), jnp.float32)])
    row2d = row.reshape(-1, K)
    col2d = col.reshape(-1, K)

    spmm_x = _make_spmm(N, MP, nnz_pad)
    spmm_xt = _make_spmm(MP, N, nnz_pad)

    rn = N * B // 128
    rm = MP * B // 128

    def a_op(vv2d):
        vv = vv2d.reshape(N, B)
        up = spmm_x(vv, col, row2d, val)
        u = _combine_partials(up.reshape(NUM_CORES, rm, 128)).reshape(MP, B)
        ap = spmm_xt(u, row, col2d, val)
        return ap.reshape(NUM_CORES, rn, 128)

    v2d = rating_matrix.T.reshape(rn, 128)

    t1, out = pl.pallas_call(
        _init_kernel,
        out_shape=(jax.ShapeDtypeStruct((rn, 128), jnp.float32),
                   jax.ShapeDtypeStruct((rn, 128), jnp.float32)),
    )(a_op(v2d), v2d)

    t0 = v2d
    for k in range(2, DEGREE + 1):
        t_new, out = pl.pallas_call(
            _make_step_kernel(_COEFFS[k]),
            out_shape=(jax.ShapeDtypeStruct((rn, 128), jnp.float32),
                       jax.ShapeDtypeStruct((rn, 128), jnp.float32)),
        )(a_op(t1), t1, t0, out)
        t0, t1 = t1, t_new

    return out.reshape(N, B).T


# speedup vs baseline: 16.7905x; 1.3073x over previous
import functools

import numpy as np
import jax
import jax.numpy as jnp
from jax import lax
from jax.experimental import pallas as pl
from jax.experimental.pallas import tpu as pltpu
from jax.experimental.pallas import tpu_sc as plsc

M = 20000
MP = 20096
N = 16384
B = 32
DEGREE = 6
ALPHA = 500.0
BETA = 0.5
LAM_MAX = 100.0
LAM_MIN = 0.0

NUM_CORES = 2
NUM_SUBCORES = 16
NUM_TILES = NUM_CORES * NUM_SUBCORES
K = 128

RPT_U = MP // NUM_SUBCORES
RPT_N = N // NUM_SUBCORES
DSL_U = MP // NUM_TILES
DSL_N = N // NUM_TILES


def _cheby_coeffs():
    exponent = 1.0 / (1.0 + BETA)
    j = np.arange(DEGREE + 1)
    theta = np.pi * (j + 0.5) / (DEGREE + 1)
    mid, half = (LAM_MAX + LAM_MIN) / 2.0, (LAM_MAX - LAM_MIN) / 2.0
    lam_nodes = mid + half * np.cos(theta)
    lam_pow = np.power(np.maximum(lam_nodes, 1e-12), exponent)
    f_nodes = lam_pow / (lam_pow + ALPHA)
    coeffs = np.zeros(DEGREE + 1)
    for k in range(DEGREE + 1):
        coeffs[k] = 2.0 / (DEGREE + 1) * np.sum(f_nodes * np.cos(k * theta))
    coeffs[0] /= 2.0
    return [float(c) for c in coeffs], float(mid), float(half)


_COEFFS, _MID, _HALF = _cheby_coeffs()


_BCAST_DN = lax.GatherDimensionNumbers(
    offset_dims=(), collapsed_slice_dims=(0,), start_index_map=(0,))


def _lane_bcast(v16, lane):
    return lax.gather(v16, jnp.full((16, 1), lane, jnp.int32), _BCAST_DN, (1,),
                      mode=lax.GatherScatterMode.PROMISE_IN_BOUNDS)


@functools.lru_cache(maxsize=None)
def _make_fused(nnz_pad):
    per_tile = nnz_pad // NUM_TILES
    nbatch = per_tile // K
    mesh = plsc.VectorSubcoreMesh(core_axis_name="c", subcore_axis_name="s")

    out_types = (
        jax.ShapeDtypeStruct((N, B), jnp.float32),
        jax.ShapeDtypeStruct((NUM_CORES, MP, B), jnp.float32),
        jax.ShapeDtypeStruct((MP, B), jnp.float32),
        jax.ShapeDtypeStruct((NUM_CORES, N, B), jnp.float32),
        jax.ShapeDtypeStruct((N, B), jnp.float32),
        jax.ShapeDtypeStruct((N, B), jnp.float32),
    )

    @functools.partial(
        pl.kernel,
        out_type=out_types,
        mesh=mesh,
        compiler_params=pltpu.CompilerParams(use_tc_tiling_on_sc=False),
        scratch_types=[
            pltpu.VMEM((nbatch, K), jnp.int32),
            pltpu.VMEM((nbatch, K), jnp.int32),
            pltpu.VMEM((per_tile,), jnp.float32),
            pltpu.VMEM((K, B), jnp.float32),
            pltpu.VMEM((K, B), jnp.float32),
            pltpu.VMEM((K, B), jnp.float32),
            pltpu.VMEM((K, B), jnp.float32),
            pltpu.VMEM((K, B), jnp.float32),
            pltpu.VMEM((K, B), jnp.float32),
            pltpu.VMEM_SHARED((MP, B), jnp.float32),
            pltpu.SemaphoreType.DMA,
            pltpu.SemaphoreType.DMA,
            pltpu.SemaphoreType.DMA,
            pltpu.SemaphoreType.DMA,
            pltpu.SemaphoreType.DMA,
            pltpu.SemaphoreType.REGULAR,
        ],
    )
    def fused(v_hbm, col2d_hbm, row2d_hbm, val_hbm,
              res_hbm, up_hbm, uc_hbm, qp_hbm, tcur_hbm, tprev_hbm,
              cidx_v, ridx_v, val_v, rows_a, rows_b, d0, d1, d2, zb, acc,
              gsa, gsb, ssa, ssb, stg, bsem):
        cid = lax.axis_index("c")
        sid = lax.axis_index("s")
        wid = cid * NUM_SUBCORES + sid
        ebase = wid * per_tile
        rbase = wid * nbatch

        def gbar():
            plsc.subcore_barrier()
            @pl.when(sid == 0)
            def _():
                pl.semaphore_signal(bsem, 1, core_index=1 - cid)
                pl.semaphore_wait(bsem, 1)
            plsc.subcore_barrier()

        pltpu.async_copy(val_hbm.at[pl.ds(ebase, per_tile)], val_v, stg)
        pltpu.async_copy(col2d_hbm.at[pl.ds(rbase, nbatch)], cidx_v, gsa)
        pltpu.async_copy(row2d_hbm.at[pl.ds(rbase, nbatch)], ridx_v, gsb)

        zero16 = jnp.zeros((16,), jnp.float32)
        def zrow(i, carry):
            zb[i, pl.ds(0, 16)] = zero16
            zb[i, pl.ds(16, 16)] = zero16
            return carry
        lax.fori_loop(0, K, zrow, 0)

        def zero_acc_slab(rpt):
            full, rem = divmod(rpt, K)
            def zchunk(z, carry):
                pltpu.sync_copy(zb, acc.at[pl.ds(sid * rpt + z * K, K)])
                return carry
            lax.fori_loop(0, full, zchunk, 0)
            if rem:
                pltpu.sync_copy(zb.at[pl.ds(0, rem)],
                                acc.at[pl.ds(sid * rpt + full * K, rem)])

        def scale(buf, i):
            @plsc.parallel_loop(0, K // 16)
            def _(g):
                val16 = val_v[pl.ds(i * K + g * 16, 16)]
                for kk in range(16):
                    e = g * 16 + kk
                    bv = _lane_bcast(val16, kk)
                    buf[e, pl.ds(0, 16)] = buf[e, pl.ds(0, 16)] * bv
                    buf[e, pl.ds(16, 16)] = buf[e, pl.ds(16, 16)] * bv

        def run_spmm(src_hbm, gidx_v, sidx_v):
            def issue_gather(i, buf, sem):
                pltpu.async_copy(src_hbm.at[gidx_v.at[i]], buf, sem)

            def issue_scatter(i, buf, sem):
                pltpu.async_copy(buf, acc.at[sidx_v.at[i]], sem, add=True)

            def wait_rows_sem(buf, sem):
                pltpu.make_async_copy(src_hbm.at[pl.ds(0, K)], buf, sem).wait()

            issue_gather(0, rows_a, gsa)
            wait_rows_sem(rows_a, gsa)
            issue_gather(1, rows_b, gsb)
            scale(rows_a, 0)
            issue_scatter(0, rows_a, ssa)

            def body(i, carry):
                @pl.when(i % 2 == 1)
                def _():
                    wait_rows_sem(rows_b, gsb)
                    wait_rows_sem(rows_a, ssa)
                    @pl.when(i + 1 < nbatch)
                    def _():
                        issue_gather(i + 1, rows_a, gsa)
                    scale(rows_b, i)
                    issue_scatter(i, rows_b, ssb)

                @pl.when(i % 2 == 0)
                def _():
                    wait_rows_sem(rows_a, gsa)
                    wait_rows_sem(rows_b, ssb)
                    @pl.when(i + 1 < nbatch)
                    def _():
                        issue_gather(i + 1, rows_b, gsb)
                    scale(rows_a, i)
                    issue_scatter(i, rows_a, ssa)

                return carry

            lax.fori_loop(1, nbatch, body, 0)
            wait_rows_sem(rows_b, ssb)

        def copy_acc_out(rpt, dst_hbm):
            full, rem = divmod(rpt, K)
            def cchunk(z, carry):
                pltpu.sync_copy(acc.at[pl.ds(sid * rpt + z * K, K)],
                                dst_hbm.at[cid, pl.ds(sid * rpt + z * K, K)])
                return carry
            lax.fori_loop(0, full, cchunk, 0)
            if rem:
                pltpu.sync_copy(acc.at[pl.ds(sid * rpt + full * K, rem)],
                                dst_hbm.at[cid, pl.ds(sid * rpt + full * K, rem)])

        def add_rows(dst, a, b, n):
            def arow(r, carry):
                dst[r, pl.ds(0, 16)] = a[r, pl.ds(0, 16)] + b[r, pl.ds(0, 16)]
                dst[r, pl.ds(16, 16)] = a[r, pl.ds(16, 16)] + b[r, pl.ds(16, 16)]
                return carry
            lax.fori_loop(0, n, arow, 0)

        def dense_combine():
            base = wid * DSL_U
            full, rem = divmod(DSL_U, K)
            for z in range(full + (1 if rem else 0)):
                cn = K if z < full else rem
                off = base + z * K
                pltpu.async_copy(up_hbm.at[0, pl.ds(off, cn)], rows_a.at[pl.ds(0, cn)], gsa)
                pltpu.async_copy(up_hbm.at[1, pl.ds(off, cn)], rows_b.at[pl.ds(0, cn)], gsb)
                pltpu.make_async_copy(up_hbm.at[0, pl.ds(off, cn)], rows_a.at[pl.ds(0, cn)], gsa).wait()
                pltpu.make_async_copy(up_hbm.at[1, pl.ds(off, cn)], rows_b.at[pl.ds(0, cn)], gsb).wait()
                add_rows(rows_a, rows_a, rows_b, cn)
                pltpu.sync_copy(rows_a.at[pl.ds(0, cn)], uc_hbm.at[pl.ds(off, cn)])

        lane_ids = lax.iota(jnp.int32, 16)
        coeffs16 = jnp.zeros((16,), jnp.float32)
        for ci, cval in enumerate(_COEFFS):
            coeffs16 = jnp.where(lane_ids == ci, jnp.float32(cval), coeffs16)

        def dense_recurrence_first():
            base = wid * DSL_N
            for z in range(DSL_N // K):
                off = base + z * K
                pltpu.async_copy(qp_hbm.at[0, pl.ds(off, K)], rows_a, gsa)
                pltpu.async_copy(qp_hbm.at[1, pl.ds(off, K)], rows_b, gsb)
                pltpu.async_copy(v_hbm.at[pl.ds(off, K)], d0, ssa)
                pltpu.make_async_copy(qp_hbm.at[0, pl.ds(off, K)], rows_a, gsa).wait()
                pltpu.make_async_copy(qp_hbm.at[1, pl.ds(off, K)], rows_b, gsb).wait()
                pltpu.make_async_copy(v_hbm.at[pl.ds(off, K)], d0, ssa).wait()
                c0, c1 = _COEFFS[0], _COEFFS[1]
                ch, cm = 1.0 / _HALF, _MID / _HALF
                def row1(r, carry):
                    for h in (0, 16):
                        a = rows_a[r, pl.ds(h, 16)] + rows_b[r, pl.ds(h, 16)]
                        vv = d0[r, pl.ds(h, 16)]
                        t1 = a * ch - vv * cm
                        rows_a[r, pl.ds(h, 16)] = t1
                        rows_b[r, pl.ds(h, 16)] = vv * c0 + t1 * c1
                    return carry
                lax.fori_loop(0, K, row1, 0)
                pltpu.sync_copy(d0, tprev_hbm.at[pl.ds(off, K)])
                pltpu.sync_copy(rows_a, tcur_hbm.at[pl.ds(off, K)])
                pltpu.sync_copy(rows_b, res_hbm.at[pl.ds(off, K)])

        def dense_recurrence_step(k):
            ckv = _lane_bcast(coeffs16, k)
            base = wid * DSL_N
            for z in range(DSL_N // K):
                off = base + z * K
                pltpu.async_copy(qp_hbm.at[0, pl.ds(off, K)], rows_a, gsa)
                pltpu.async_copy(qp_hbm.at[1, pl.ds(off, K)], rows_b, gsb)
                pltpu.async_copy(tcur_hbm.at[pl.ds(off, K)], d0, ssa)
                pltpu.async_copy(tprev_hbm.at[pl.ds(off, K)], d1, ssb)
                pltpu.async_copy(res_hbm.at[pl.ds(off, K)], d2, stg)
                pltpu.make_async_copy(qp_hbm.at[0, pl.ds(off, K)], rows_a, gsa).wait()
                pltpu.make_async_copy(qp_hbm.at[1, pl.ds(off, K)], rows_b, gsb).wait()
                pltpu.make_async_copy(tcur_hbm.at[pl.ds(off, K)], d0, ssa).wait()
                pltpu.make_async_copy(tprev_hbm.at[pl.ds(off, K)], d1, ssb).wait()
                pltpu.make_async_copy(res_hbm.at[pl.ds(off, K)], d2, stg).wait()
                c2h, c2m = 2.0 / _HALF, 2.0 * _MID / _HALF
                def rowk(r, carry):
                    for h in (0, 16):
                        a = rows_a[r, pl.ds(h, 16)] + rows_b[r, pl.ds(h, 16)]
                        tn = a * c2h - d0[r, pl.ds(h, 16)] * c2m - d1[r, pl.ds(h, 16)]
                        rows_a[r, pl.ds(h, 16)] = tn
                        rows_b[r, pl.ds(h, 16)] = d2[r, pl.ds(h, 16)] + tn * ckv
                    return carry
                lax.fori_loop(0, K, rowk, 0)
                pltpu.sync_copy(d0, tprev_hbm.at[pl.ds(off, K)])
                pltpu.sync_copy(rows_a, tcur_hbm.at[pl.ds(off, K)])
                pltpu.sync_copy(rows_b, res_hbm.at[pl.ds(off, K)])

        pltpu.make_async_copy(val_hbm.at[pl.ds(ebase, per_tile)], val_v, stg).wait()
        pltpu.make_async_copy(col2d_hbm.at[pl.ds(rbase, nbatch)], cidx_v, gsa).wait()
        pltpu.make_async_copy(row2d_hbm.at[pl.ds(rbase, nbatch)], ridx_v, gsb).wait()

        def iteration(k, src_hbm, dense_phase):
            zero_acc_slab(RPT_U)
            plsc.subcore_barrier()
            run_spmm(src_hbm, cidx_v, ridx_v)
            plsc.subcore_barrier()
            copy_acc_out(RPT_U, up_hbm)
            gbar()
            dense_combine()
            zero_acc_slab(RPT_N)
            gbar()
            run_spmm(uc_hbm, ridx_v, cidx_v)
            plsc.subcore_barrier()
            copy_acc_out(RPT_N, qp_hbm)
            gbar()
            dense_phase(k)
            gbar()

        iteration(1, v_hbm, lambda k: dense_recurrence_first())

        def body(k, carry):
            iteration(k, tcur_hbm, dense_recurrence_step)
            return carry
        lax.fori_loop(2, DEGREE + 1, body, 0)

    return fused


def kernel(rating_matrix, x_row, x_col, x_val):
    nnz = x_row.shape[0]
    nnz_pad = -(-nnz // (NUM_TILES * K * 2)) * (NUM_TILES * K * 2)
    pad = nnz_pad - nnz

    row = jnp.concatenate([x_row.astype(jnp.int32), jnp.zeros((pad,), jnp.int32)])
    col = jnp.concatenate([x_col.astype(jnp.int32), jnp.zeros((pad,), jnp.int32)])
    val = jnp.concatenate([x_val.astype(jnp.float32), jnp.zeros((pad,), jnp.float32)])
    row2d = row.reshape(-1, K)
    col2d = col.reshape(-1, K)

    fused = _make_fused(nnz_pad)
    v = rating_matrix.T.reshape(N, B)
    res = fused(v, col2d, row2d, val)[0]
    return res.T


# speedup vs baseline: 20.6962x; 1.2326x over previous
import functools

import numpy as np
import jax
import jax.numpy as jnp
from jax import lax
from jax.experimental import pallas as pl
from jax.experimental.pallas import tpu as pltpu
from jax.experimental.pallas import tpu_sc as plsc

M = 20000
MP = 20096
N = 16384
B = 32
DEGREE = 6
ALPHA = 500.0
BETA = 0.5
LAM_MAX = 100.0
LAM_MIN = 0.0

NUM_CORES = 2
NUM_SUBCORES = 16
NUM_TILES = NUM_CORES * NUM_SUBCORES
K = 128

RPT_U = MP // NUM_SUBCORES
RPT_N = N // NUM_SUBCORES
DSL_U = MP // NUM_TILES
DSL_N = N // NUM_TILES


def _cheby_coeffs():
    exponent = 1.0 / (1.0 + BETA)
    j = np.arange(DEGREE + 1)
    theta = np.pi * (j + 0.5) / (DEGREE + 1)
    mid, half = (LAM_MAX + LAM_MIN) / 2.0, (LAM_MAX - LAM_MIN) / 2.0
    lam_nodes = mid + half * np.cos(theta)
    lam_pow = np.power(np.maximum(lam_nodes, 1e-12), exponent)
    f_nodes = lam_pow / (lam_pow + ALPHA)
    coeffs = np.zeros(DEGREE + 1)
    for k in range(DEGREE + 1):
        coeffs[k] = 2.0 / (DEGREE + 1) * np.sum(f_nodes * np.cos(k * theta))
    coeffs[0] /= 2.0
    return [float(c) for c in coeffs], float(mid), float(half)


_COEFFS, _MID, _HALF = _cheby_coeffs()


_BCAST_DN = lax.GatherDimensionNumbers(
    offset_dims=(), collapsed_slice_dims=(0,), start_index_map=(0,))


def _lane_bcast(v16, lane):
    return lax.gather(v16, jnp.full((16, 1), lane, jnp.int32), _BCAST_DN, (1,),
                      mode=lax.GatherScatterMode.PROMISE_IN_BOUNDS)


@functools.lru_cache(maxsize=None)
def _make_fused(nnz_pad):
    per_tile = nnz_pad // NUM_TILES
    nbatch = per_tile // K
    mesh = plsc.VectorSubcoreMesh(core_axis_name="c", subcore_axis_name="s")

    out_types = (
        jax.ShapeDtypeStruct((N, B), jnp.float32),
        jax.ShapeDtypeStruct((NUM_CORES, MP, B), jnp.float32),
        jax.ShapeDtypeStruct((MP, B), jnp.float32),
        jax.ShapeDtypeStruct((NUM_CORES, N, B), jnp.float32),
        jax.ShapeDtypeStruct((N, B), jnp.float32),
        jax.ShapeDtypeStruct((N, B), jnp.float32),
    )

    @functools.partial(
        pl.kernel,
        out_type=out_types,
        mesh=mesh,
        compiler_params=pltpu.CompilerParams(use_tc_tiling_on_sc=False),
        scratch_types=[
            pltpu.VMEM((nbatch, K), jnp.int32),
            pltpu.VMEM((nbatch, K), jnp.int32),
            pltpu.VMEM((per_tile,), jnp.float32),
            pltpu.VMEM((K, B), jnp.float32),
            pltpu.VMEM((K, B), jnp.float32),
            pltpu.VMEM((K, B), jnp.float32),
            pltpu.VMEM((K, B), jnp.float32),
            pltpu.VMEM((K, B), jnp.float32),
            pltpu.VMEM((K, B), jnp.float32),
            pltpu.VMEM_SHARED((MP, B), jnp.float32),
            pltpu.SemaphoreType.DMA,
            pltpu.SemaphoreType.DMA,
            pltpu.SemaphoreType.DMA,
            pltpu.SemaphoreType.DMA,
            pltpu.SemaphoreType.DMA,
            pltpu.SemaphoreType.REGULAR,
        ],
    )
    def fused(v_hbm, col2d_hbm, row2d_hbm, val_hbm,
              res_hbm, up_hbm, uc_hbm, qp_hbm, tcur_hbm, tprev_hbm,
              cidx_v, ridx_v, val_v, rows_a, rows_b, d0, d1, d2, zb, acc,
              gsa, gsb, ssa, ssb, stg, bsem):
        cid = lax.axis_index("c")
        sid = lax.axis_index("s")
        wid = cid * NUM_SUBCORES + sid
        ebase = wid * per_tile
        rbase = wid * nbatch

        def gbar():
            plsc.subcore_barrier()
            @pl.when(sid == 0)
            def _():
                pl.semaphore_signal(bsem, 1, core_index=1 - cid)
                pl.semaphore_wait(bsem, 1)
            plsc.subcore_barrier()

        pltpu.async_copy(val_hbm.at[pl.ds(ebase, per_tile)], val_v, stg)
        pltpu.async_copy(col2d_hbm.at[pl.ds(rbase, nbatch)], cidx_v, gsa)
        pltpu.async_copy(row2d_hbm.at[pl.ds(rbase, nbatch)], ridx_v, gsb)

        zero16 = jnp.zeros((16,), jnp.float32)
        def zrow(i, carry):
            zb[i, pl.ds(0, 16)] = zero16
            zb[i, pl.ds(16, 16)] = zero16
            return carry
        lax.fori_loop(0, K, zrow, 0)

        def zero_acc_slab(rpt):
            full, rem = divmod(rpt, K)
            def zchunk(z, carry):
                pltpu.sync_copy(zb, acc.at[pl.ds(sid * rpt + z * K, K)])
                return carry
            lax.fori_loop(0, full, zchunk, 0)
            if rem:
                pltpu.sync_copy(zb.at[pl.ds(0, rem)],
                                acc.at[pl.ds(sid * rpt + full * K, rem)])

        def scale(buf, i):
            @plsc.parallel_loop(0, K // 16)
            def _(g):
                val16 = val_v[pl.ds(i * K + g * 16, 16)]
                for kk in range(16):
                    e = g * 16 + kk
                    bv = _lane_bcast(val16, kk)
                    buf[e, pl.ds(0, 16)] = buf[e, pl.ds(0, 16)] * bv
                    buf[e, pl.ds(16, 16)] = buf[e, pl.ds(16, 16)] * bv

        def run_spmm(src_hbm, gidx_v, sidx_v):
            def issue_gather(i, buf, sem):
                pltpu.async_copy(src_hbm.at[gidx_v.at[i]], buf, sem)

            def issue_scatter(i, buf, sem):
                pltpu.async_copy(buf, acc.at[sidx_v.at[i]], sem, add=True)

            def wait_rows_sem(buf, sem):
                pltpu.make_async_copy(src_hbm.at[pl.ds(0, K)], buf, sem).wait()

            issue_gather(0, rows_a, gsa)
            issue_gather(1, rows_b, gsb)
            wait_rows_sem(rows_a, gsa)
            scale(rows_a, 0)
            issue_scatter(0, rows_a, ssa)

            def body(i, carry):
                @pl.when(i % 2 == 1)
                def _():
                    wait_rows_sem(rows_a, ssa)
                    @pl.when(i + 1 < nbatch)
                    def _():
                        issue_gather(i + 1, rows_a, gsa)
                    wait_rows_sem(rows_b, gsb)
                    scale(rows_b, i)
                    issue_scatter(i, rows_b, ssb)

                @pl.when(i % 2 == 0)
                def _():
                    wait_rows_sem(rows_b, ssb)
                    @pl.when(i + 1 < nbatch)
                    def _():
                        issue_gather(i + 1, rows_b, gsb)
                    wait_rows_sem(rows_a, gsa)
                    scale(rows_a, i)
                    issue_scatter(i, rows_a, ssa)

                return carry

            lax.fori_loop(1, nbatch, body, 0)
            wait_rows_sem(rows_b, ssb)

        def copy_acc_out(rpt, dst_hbm):
            full, rem = divmod(rpt, K)
            def cchunk(z, carry):
                pltpu.sync_copy(acc.at[pl.ds(sid * rpt + z * K, K)],
                                dst_hbm.at[cid, pl.ds(sid * rpt + z * K, K)])
                return carry
            lax.fori_loop(0, full, cchunk, 0)
            if rem:
                pltpu.sync_copy(acc.at[pl.ds(sid * rpt + full * K, rem)],
                                dst_hbm.at[cid, pl.ds(sid * rpt + full * K, rem)])

        def add_rows(dst, a, b, n):
            def arow(r, carry):
                dst[r, pl.ds(0, 16)] = a[r, pl.ds(0, 16)] + b[r, pl.ds(0, 16)]
                dst[r, pl.ds(16, 16)] = a[r, pl.ds(16, 16)] + b[r, pl.ds(16, 16)]
                return carry
            lax.fori_loop(0, n, arow, 0)

        def dense_combine():
            base = wid * DSL_U
            full, rem = divmod(DSL_U, K)
            for z in range(full + (1 if rem else 0)):
                cn = K if z < full else rem
                off = base + z * K
                pltpu.async_copy(up_hbm.at[0, pl.ds(off, cn)], rows_a.at[pl.ds(0, cn)], gsa)
                pltpu.async_copy(up_hbm.at[1, pl.ds(off, cn)], rows_b.at[pl.ds(0, cn)], gsb)
                pltpu.make_async_copy(up_hbm.at[0, pl.ds(off, cn)], rows_a.at[pl.ds(0, cn)], gsa).wait()
                pltpu.make_async_copy(up_hbm.at[1, pl.ds(off, cn)], rows_b.at[pl.ds(0, cn)], gsb).wait()
                add_rows(rows_a, rows_a, rows_b, cn)
                pltpu.sync_copy(rows_a.at[pl.ds(0, cn)], uc_hbm.at[pl.ds(off, cn)])

        lane_ids = lax.iota(jnp.int32, 16)
        coeffs16 = jnp.zeros((16,), jnp.float32)
        for ci, cval in enumerate(_COEFFS):
            coeffs16 = jnp.where(lane_ids == ci, jnp.float32(cval), coeffs16)

        def dense_recurrence_first():
            base = wid * DSL_N
            for z in range(DSL_N // K):
                off = base + z * K
                pltpu.async_copy(qp_hbm.at[0, pl.ds(off, K)], rows_a, gsa)
                pltpu.async_copy(qp_hbm.at[1, pl.ds(off, K)], rows_b, gsb)
                pltpu.async_copy(v_hbm.at[pl.ds(off, K)], d0, ssa)
                pltpu.make_async_copy(qp_hbm.at[0, pl.ds(off, K)], rows_a, gsa).wait()
                pltpu.make_async_copy(qp_hbm.at[1, pl.ds(off, K)], rows_b, gsb).wait()
                pltpu.make_async_copy(v_hbm.at[pl.ds(off, K)], d0, ssa).wait()
                c0, c1 = _COEFFS[0], _COEFFS[1]
                ch, cm = 1.0 / _HALF, _MID / _HALF
                def row1(r, carry):
                    for h in (0, 16):
                        a = rows_a[r, pl.ds(h, 16)] + rows_b[r, pl.ds(h, 16)]
                        vv = d0[r, pl.ds(h, 16)]
                        t1 = a * ch - vv * cm
                        rows_a[r, pl.ds(h, 16)] = t1
                        rows_b[r, pl.ds(h, 16)] = vv * c0 + t1 * c1
                    return carry
                lax.fori_loop(0, K, row1, 0)
                pltpu.sync_copy(d0, tprev_hbm.at[pl.ds(off, K)])
                pltpu.sync_copy(rows_a, tcur_hbm.at[pl.ds(off, K)])
                pltpu.sync_copy(rows_b, res_hbm.at[pl.ds(off, K)])

        def dense_recurrence_step(k):
            ckv = _lane_bcast(coeffs16, k)
            base = wid * DSL_N
            for z in range(DSL_N // K):
                off = base + z * K
                pltpu.async_copy(qp_hbm.at[0, pl.ds(off, K)], rows_a, gsa)
                pltpu.async_copy(qp_hbm.at[1, pl.ds(off, K)], rows_b, gsb)
                pltpu.async_copy(tcur_hbm.at[pl.ds(off, K)], d0, ssa)
                pltpu.async_copy(tprev_hbm.at[pl.ds(off, K)], d1, ssb)
                pltpu.async_copy(res_hbm.at[pl.ds(off, K)], d2, stg)
                pltpu.make_async_copy(qp_hbm.at[0, pl.ds(off, K)], rows_a, gsa).wait()
                pltpu.make_async_copy(qp_hbm.at[1, pl.ds(off, K)], rows_b, gsb).wait()
                pltpu.make_async_copy(tcur_hbm.at[pl.ds(off, K)], d0, ssa).wait()
                pltpu.make_async_copy(tprev_hbm.at[pl.ds(off, K)], d1, ssb).wait()
                pltpu.make_async_copy(res_hbm.at[pl.ds(off, K)], d2, stg).wait()
                c2h, c2m = 2.0 / _HALF, 2.0 * _MID / _HALF
                def rowk(r, carry):
                    for h in (0, 16):
                        a = rows_a[r, pl.ds(h, 16)] + rows_b[r, pl.ds(h, 16)]
                        tn = a * c2h - d0[r, pl.ds(h, 16)] * c2m - d1[r, pl.ds(h, 16)]
                        rows_a[r, pl.ds(h, 16)] = tn
                        rows_b[r, pl.ds(h, 16)] = d2[r, pl.ds(h, 16)] + tn * ckv
                    return carry
                lax.fori_loop(0, K, rowk, 0)
                pltpu.sync_copy(d0, tprev_hbm.at[pl.ds(off, K)])
                pltpu.sync_copy(rows_a, tcur_hbm.at[pl.ds(off, K)])
                pltpu.sync_copy(rows_b, res_hbm.at[pl.ds(off, K)])

        pltpu.make_async_copy(val_hbm.at[pl.ds(ebase, per_tile)], val_v, stg).wait()
        pltpu.make_async_copy(col2d_hbm.at[pl.ds(rbase, nbatch)], cidx_v, gsa).wait()
        pltpu.make_async_copy(row2d_hbm.at[pl.ds(rbase, nbatch)], ridx_v, gsb).wait()

        def iteration(k, src_hbm, dense_phase):
            zero_acc_slab(RPT_U)
            plsc.subcore_barrier()
            run_spmm(src_hbm, cidx_v, ridx_v)
            plsc.subcore_barrier()
            copy_acc_out(RPT_U, up_hbm)
            gbar()
            dense_combine()
            zero_acc_slab(RPT_N)
            gbar()
            run_spmm(uc_hbm, ridx_v, cidx_v)
            plsc.subcore_barrier()
            copy_acc_out(RPT_N, qp_hbm)
            gbar()
            dense_phase(k)
            gbar()

        iteration(1, v_hbm, lambda k: dense_recurrence_first())

        def body(k, carry):
            iteration(k, tcur_hbm, dense_recurrence_step)
            return carry
        lax.fori_loop(2, DEGREE + 1, body, 0)

    return fused


def kernel(rating_matrix, x_row, x_col, x_val):
    nnz = x_row.shape[0]
    nnz_pad = -(-nnz // (NUM_TILES * K * 2)) * (NUM_TILES * K * 2)
    pad = nnz_pad - nnz

    row = jnp.concatenate([x_row.astype(jnp.int32), jnp.zeros((pad,), jnp.int32)])
    col = jnp.concatenate([x_col.astype(jnp.int32), jnp.zeros((pad,), jnp.int32)])
    val = jnp.concatenate([x_val.astype(jnp.float32), jnp.zeros((pad,), jnp.float32)])
    row2d = row.reshape(-1, K)
    col2d = col.reshape(-1, K)

    fused = _make_fused(nnz_pad)
    v = rating_matrix.T.reshape(N, B)
    res = fused(v, col2d, row2d, val)[0]
    return res.T


# speedup vs baseline: 33.7210x; 1.6293x over previous
import functools

import numpy as np
import jax
import jax.numpy as jnp
from jax import lax
from jax.experimental import pallas as pl
from jax.experimental.pallas import tpu as pltpu
from jax.experimental.pallas import tpu_sc as plsc

M = 20000
MP = 20096
N = 16384
B = 32
DEGREE = 6
ALPHA = 500.0
BETA = 0.5
LAM_MAX = 100.0
LAM_MIN = 0.0

NUM_CORES = 2
NUM_SUBCORES = 16
NUM_TILES = NUM_CORES * NUM_SUBCORES
K = 128
RING = 8

RPT_U = MP // NUM_SUBCORES
RPT_N = N // NUM_SUBCORES
DSL_U = MP // NUM_TILES
DSL_N = N // NUM_TILES


def _cheby_coeffs():
    exponent = 1.0 / (1.0 + BETA)
    j = np.arange(DEGREE + 1)
    theta = np.pi * (j + 0.5) / (DEGREE + 1)
    mid, half = (LAM_MAX + LAM_MIN) / 2.0, (LAM_MAX - LAM_MIN) / 2.0
    lam_nodes = mid + half * np.cos(theta)
    lam_pow = np.power(np.maximum(lam_nodes, 1e-12), exponent)
    f_nodes = lam_pow / (lam_pow + ALPHA)
    coeffs = np.zeros(DEGREE + 1)
    for k in range(DEGREE + 1):
        coeffs[k] = 2.0 / (DEGREE + 1) * np.sum(f_nodes * np.cos(k * theta))
    coeffs[0] /= 2.0
    return [float(c) for c in coeffs], float(mid), float(half)


_COEFFS, _MID, _HALF = _cheby_coeffs()


_BCAST_DN = lax.GatherDimensionNumbers(
    offset_dims=(), collapsed_slice_dims=(0,), start_index_map=(0,))


def _lane_bcast(v16, lane):
    return lax.gather(v16, jnp.full((16, 1), lane, jnp.int32), _BCAST_DN, (1,),
                      mode=lax.GatherScatterMode.PROMISE_IN_BOUNDS)


@functools.lru_cache(maxsize=None)
def _make_fused(nnz_pad):
    per_tile = nnz_pad // NUM_TILES
    nbatch = per_tile // K
    mesh = plsc.VectorSubcoreMesh(core_axis_name="c", subcore_axis_name="s")

    out_types = (
        jax.ShapeDtypeStruct((N, B), jnp.float32),
        jax.ShapeDtypeStruct((NUM_CORES, MP, B), jnp.float32),
        jax.ShapeDtypeStruct((NUM_CORES, N, B), jnp.float32),
        jax.ShapeDtypeStruct((N, B), jnp.float32),
        jax.ShapeDtypeStruct((N, B), jnp.float32),
    )

    @functools.partial(
        pl.kernel,
        out_type=out_types,
        mesh=mesh,
        compiler_params=pltpu.CompilerParams(use_tc_tiling_on_sc=False),
        scratch_types=[
            pltpu.VMEM((RING, 2, K), jnp.int32),
            pltpu.VMEM((per_tile,), jnp.float32),
            pltpu.VMEM((K, B), jnp.float32),
            pltpu.VMEM((K, B), jnp.float32),
            pltpu.VMEM((K, B), jnp.float32),
            pltpu.VMEM((K, B), jnp.float32),
            pltpu.VMEM((K, B), jnp.float32),
            pltpu.VMEM((K, B), jnp.float32),
            pltpu.VMEM_SHARED((MP, B), jnp.float32),
            pltpu.VMEM_SHARED((MP, B), jnp.float32),
            pltpu.SemaphoreType.DMA,
            pltpu.SemaphoreType.DMA,
            pltpu.SemaphoreType.DMA,
            pltpu.SemaphoreType.DMA,
            pltpu.SemaphoreType.DMA,
            pltpu.SemaphoreType.DMA,
            pltpu.SemaphoreType.REGULAR,
        ],
    )
    def fused(v_hbm, idx2_hbm, val_hbm,
              res_hbm, up_hbm, qp_hbm, tcur_hbm, tprev_hbm,
              ring, val_v, rows_a, rows_b, d0, d1, d2, zb, acc,
              src_sp, gsa, gsb, ssa, ssb, stg, psem, bsem):
        cid = lax.axis_index("c")
        sid = lax.axis_index("s")
        wid = cid * NUM_SUBCORES + sid
        ebase = wid * per_tile
        rbase = wid * nbatch

        def gbar():
            plsc.subcore_barrier()
            @pl.when(sid == 0)
            def _():
                pl.semaphore_signal(bsem, 1, core_index=1 - cid)
                pl.semaphore_wait(bsem, 1)
            plsc.subcore_barrier()

        pltpu.async_copy(val_hbm.at[pl.ds(ebase, per_tile)], val_v, stg)

        zero16 = jnp.zeros((16,), jnp.float32)
        def zrow(i, carry):
            zb[i, pl.ds(0, 16)] = zero16
            zb[i, pl.ds(16, 16)] = zero16
            return carry
        lax.fori_loop(0, K, zrow, 0)

        def zero_acc_slab(rpt):
            full, rem = divmod(rpt, K)
            def zchunk(z, carry):
                pltpu.sync_copy(zb, acc.at[pl.ds(sid * rpt + z * K, K)])
                return carry
            lax.fori_loop(0, full, zchunk, 0)
            if rem:
                pltpu.sync_copy(zb.at[pl.ds(0, rem)],
                                acc.at[pl.ds(sid * rpt + full * K, rem)])

        def scale(buf, i):
            @plsc.parallel_loop(0, K // 16)
            def _(g):
                val16 = val_v[pl.ds(i * K + g * 16, 16)]
                for kk in range(16):
                    e = g * 16 + kk
                    bv = _lane_bcast(val16, kk)
                    buf[e, pl.ds(0, 16)] = buf[e, pl.ds(0, 16)] * bv
                    buf[e, pl.ds(16, 16)] = buf[e, pl.ds(16, 16)] * bv

        def stage_rows(src_hbm, nrows):
            per = nrows // NUM_SUBCORES
            start = sid * per
            full, rem = divmod(per, K)
            for z in range(full):
                pltpu.async_copy(src_hbm.at[pl.ds(start + z * K, K)],
                                 src_sp.at[pl.ds(start + z * K, K)], stg)
            if rem:
                pltpu.async_copy(src_hbm.at[pl.ds(start + full * K, rem)],
                                 src_sp.at[pl.ds(start + full * K, rem)], stg)
            for z in range(full):
                pltpu.make_async_copy(src_hbm.at[pl.ds(start + z * K, K)],
                                      src_sp.at[pl.ds(start + z * K, K)], stg).wait()
            if rem:
                pltpu.make_async_copy(src_hbm.at[pl.ds(start + full * K, rem)],
                                      src_sp.at[pl.ds(start + full * K, rem)], stg).wait()

        def prefetch_idx(j):
            pltpu.async_copy(idx2_hbm.at[rbase + j], ring.at[j % RING], psem)

        def wait_idx():
            pltpu.make_async_copy(idx2_hbm.at[0], ring.at[0], psem).wait()

        def run_spmm(gsel, ssel):
            def issue_gather(i, buf, sem):
                pltpu.async_copy(src_sp.at[ring.at[i % RING, gsel]], buf, sem)

            def issue_scatter(i, buf, sem):
                pltpu.async_copy(buf, acc.at[ring.at[i % RING, ssel]], sem,
                                 add=True)

            def wait_rows_sem(buf, sem):
                pltpu.make_async_copy(src_sp.at[pl.ds(0, K)], buf, sem).wait()

            for j in range(min(RING, nbatch)):
                prefetch_idx(j)
            wait_idx()
            wait_idx()
            issue_gather(0, rows_a, gsa)
            issue_gather(1, rows_b, gsb)
            wait_rows_sem(rows_a, gsa)
            scale(rows_a, 0)
            issue_scatter(0, rows_a, ssa)

            def body(i, carry):
                @pl.when(i % 2 == 1)
                def _():
                    wait_rows_sem(rows_a, ssa)
                    @pl.when(i + RING - 1 < nbatch)
                    def _():
                        prefetch_idx(i + RING - 1)
                    @pl.when(i + 1 < nbatch)
                    def _():
                        wait_idx()
                        issue_gather(i + 1, rows_a, gsa)
                    wait_rows_sem(rows_b, gsb)
                    scale(rows_b, i)
                    issue_scatter(i, rows_b, ssb)

                @pl.when(i % 2 == 0)
                def _():
                    wait_rows_sem(rows_b, ssb)
                    @pl.when(i + RING - 1 < nbatch)
                    def _():
                        prefetch_idx(i + RING - 1)
                    @pl.when(i + 1 < nbatch)
                    def _():
                        wait_idx()
                        issue_gather(i + 1, rows_b, gsb)
                    wait_rows_sem(rows_a, gsa)
                    scale(rows_a, i)
                    issue_scatter(i, rows_a, ssa)

                return carry

            lax.fori_loop(1, nbatch, body, 0)
            wait_rows_sem(rows_b, ssb)

        def copy_acc_out(rpt, dst_hbm):
            full, rem = divmod(rpt, K)
            def cchunk(z, carry):
                pltpu.sync_copy(acc.at[pl.ds(sid * rpt + z * K, K)],
                                dst_hbm.at[cid, pl.ds(sid * rpt + z * K, K)])
                return carry
            lax.fori_loop(0, full, cchunk, 0)
            if rem:
                pltpu.sync_copy(acc.at[pl.ds(sid * rpt + full * K, rem)],
                                dst_hbm.at[cid, pl.ds(sid * rpt + full * K, rem)])

        def add_rows(dst, a, b, n):
            def arow(r, carry):
                dst[r, pl.ds(0, 16)] = a[r, pl.ds(0, 16)] + b[r, pl.ds(0, 16)]
                dst[r, pl.ds(16, 16)] = a[r, pl.ds(16, 16)] + b[r, pl.ds(16, 16)]
                return carry
            lax.fori_loop(0, n, arow, 0)

        def dense_combine():
            base = sid * RPT_U
            full, rem = divmod(RPT_U, K)
            for z in range(full + (1 if rem else 0)):
                cn = K if z < full else rem
                off = base + z * K
                pltpu.async_copy(up_hbm.at[0, pl.ds(off, cn)], rows_a.at[pl.ds(0, cn)], gsa)
                pltpu.async_copy(up_hbm.at[1, pl.ds(off, cn)], rows_b.at[pl.ds(0, cn)], gsb)
                pltpu.make_async_copy(up_hbm.at[0, pl.ds(off, cn)], rows_a.at[pl.ds(0, cn)], gsa).wait()
                pltpu.make_async_copy(up_hbm.at[1, pl.ds(off, cn)], rows_b.at[pl.ds(0, cn)], gsb).wait()
                add_rows(rows_a, rows_a, rows_b, cn)
                pltpu.sync_copy(rows_a.at[pl.ds(0, cn)], src_sp.at[pl.ds(off, cn)])

        lane_ids = lax.iota(jnp.int32, 16)
        coeffs16 = jnp.zeros((16,), jnp.float32)
        for ci, cval in enumerate(_COEFFS):
            coeffs16 = jnp.where(lane_ids == ci, jnp.float32(cval), coeffs16)

        def dense_recurrence_first():
            base = wid * DSL_N
            for z in range(DSL_N // K):
                off = base + z * K
                pltpu.async_copy(qp_hbm.at[0, pl.ds(off, K)], rows_a, gsa)
                pltpu.async_copy(qp_hbm.at[1, pl.ds(off, K)], rows_b, gsb)
                pltpu.async_copy(v_hbm.at[pl.ds(off, K)], d0, ssa)
                pltpu.make_async_copy(qp_hbm.at[0, pl.ds(off, K)], rows_a, gsa).wait()
                pltpu.make_async_copy(qp_hbm.at[1, pl.ds(off, K)], rows_b, gsb).wait()
                pltpu.make_async_copy(v_hbm.at[pl.ds(off, K)], d0, ssa).wait()
                c0, c1 = _COEFFS[0], _COEFFS[1]
                ch, cm = 1.0 / _HALF, _MID / _HALF
                def row1(r, carry):
                    for h in (0, 16):
                        a = rows_a[r, pl.ds(h, 16)] + rows_b[r, pl.ds(h, 16)]
                        vv = d0[r, pl.ds(h, 16)]
                        t1 = a * ch - vv * cm
                        rows_a[r, pl.ds(h, 16)] = t1
                        rows_b[r, pl.ds(h, 16)] = vv * c0 + t1 * c1
                    return carry
                lax.fori_loop(0, K, row1, 0)
                pltpu.sync_copy(d0, tprev_hbm.at[pl.ds(off, K)])
                pltpu.sync_copy(rows_a, tcur_hbm.at[pl.ds(off, K)])
                pltpu.sync_copy(rows_b, res_hbm.at[pl.ds(off, K)])

        def dense_recurrence_step(k):
            ckv = _lane_bcast(coeffs16, k)
            base = wid * DSL_N
            for z in range(DSL_N // K):
                off = base + z * K
                pltpu.async_copy(qp_hbm.at[0, pl.ds(off, K)], rows_a, gsa)
                pltpu.async_copy(qp_hbm.at[1, pl.ds(off, K)], rows_b, gsb)
                pltpu.async_copy(tcur_hbm.at[pl.ds(off, K)], d0, ssa)
                pltpu.async_copy(tprev_hbm.at[pl.ds(off, K)], d1, ssb)
                pltpu.async_copy(res_hbm.at[pl.ds(off, K)], d2, stg)
                pltpu.make_async_copy(qp_hbm.at[0, pl.ds(off, K)], rows_a, gsa).wait()
                pltpu.make_async_copy(qp_hbm.at[1, pl.ds(off, K)], rows_b, gsb).wait()
                pltpu.make_async_copy(tcur_hbm.at[pl.ds(off, K)], d0, ssa).wait()
                pltpu.make_async_copy(tprev_hbm.at[pl.ds(off, K)], d1, ssb).wait()
                pltpu.make_async_copy(res_hbm.at[pl.ds(off, K)], d2, stg).wait()
                c2h, c2m = 2.0 / _HALF, 2.0 * _MID / _HALF
                def rowk(r, carry):
                    for h in (0, 16):
                        a = rows_a[r, pl.ds(h, 16)] + rows_b[r, pl.ds(h, 16)]
                        tn = a * c2h - d0[r, pl.ds(h, 16)] * c2m - d1[r, pl.ds(h, 16)]
                        rows_a[r, pl.ds(h, 16)] = tn
                        rows_b[r, pl.ds(h, 16)] = d2[r, pl.ds(h, 16)] + tn * ckv
                    return carry
                lax.fori_loop(0, K, rowk, 0)
                pltpu.sync_copy(d0, tprev_hbm.at[pl.ds(off, K)])
                pltpu.sync_copy(rows_a, tcur_hbm.at[pl.ds(off, K)])
                pltpu.sync_copy(rows_b, res_hbm.at[pl.ds(off, K)])

        pltpu.make_async_copy(val_hbm.at[pl.ds(ebase, per_tile)], val_v, stg).wait()

        def iteration(k, src_hbm, dense_phase):
            zero_acc_slab(RPT_U)
            stage_rows(src_hbm, N)
            plsc.subcore_barrier()
            run_spmm(0, 1)
            plsc.subcore_barrier()
            copy_acc_out(RPT_U, up_hbm)
            gbar()
            dense_combine()
            zero_acc_slab(RPT_N)
            plsc.subcore_barrier()
            run_spmm(1, 0)
            plsc.subcore_barrier()
            copy_acc_out(RPT_N, qp_hbm)
            gbar()
            dense_phase(k)
            gbar()

        iteration(1, v_hbm, lambda k: dense_recurrence_first())

        def body(k, carry):
            iteration(k, tcur_hbm, dense_recurrence_step)
            return carry
        lax.fori_loop(2, DEGREE + 1, body, 0)

    return fused


def kernel(rating_matrix, x_row, x_col, x_val):
    nnz = x_row.shape[0]
    nnz_pad = -(-nnz // (NUM_TILES * K * 2)) * (NUM_TILES * K * 2)
    pad = nnz_pad - nnz

    row = jnp.concatenate([x_row.astype(jnp.int32), jnp.zeros((pad,), jnp.int32)])
    col = jnp.concatenate([x_col.astype(jnp.int32), jnp.zeros((pad,), jnp.int32)])
    val = jnp.concatenate([x_val.astype(jnp.float32), jnp.zeros((pad,), jnp.float32)])
    idx2 = jnp.stack([col.reshape(-1, K), row.reshape(-1, K)], axis=1)

    fused = _make_fused(nnz_pad)
    v = rating_matrix.T.reshape(N, B)
    res = fused(v, idx2, val)[0]
    return res.T


# speedup vs baseline: 36.2286x; 1.0744x over previous
import functools

import numpy as np
import jax
import jax.numpy as jnp
from jax import lax
from jax.experimental import pallas as pl
from jax.experimental.pallas import tpu as pltpu
from jax.experimental.pallas import tpu_sc as plsc

M = 20000
MP = 20096
N = 16384
B = 32
DEGREE = 6
ALPHA = 500.0
BETA = 0.5
LAM_MAX = 100.0
LAM_MIN = 0.0

NUM_CORES = 2
NUM_SUBCORES = 16
NUM_TILES = NUM_CORES * NUM_SUBCORES
K = 128
RING = 8

RPT_U = MP // NUM_SUBCORES
RPT_N = N // NUM_SUBCORES
DSL_U = MP // NUM_TILES
DSL_N = N // NUM_TILES


def _cheby_coeffs():
    exponent = 1.0 / (1.0 + BETA)
    j = np.arange(DEGREE + 1)
    theta = np.pi * (j + 0.5) / (DEGREE + 1)
    mid, half = (LAM_MAX + LAM_MIN) / 2.0, (LAM_MAX - LAM_MIN) / 2.0
    lam_nodes = mid + half * np.cos(theta)
    lam_pow = np.power(np.maximum(lam_nodes, 1e-12), exponent)
    f_nodes = lam_pow / (lam_pow + ALPHA)
    coeffs = np.zeros(DEGREE + 1)
    for k in range(DEGREE + 1):
        coeffs[k] = 2.0 / (DEGREE + 1) * np.sum(f_nodes * np.cos(k * theta))
    coeffs[0] /= 2.0
    return [float(c) for c in coeffs], float(mid), float(half)


_COEFFS, _MID, _HALF = _cheby_coeffs()


_BCAST_DN = lax.GatherDimensionNumbers(
    offset_dims=(), collapsed_slice_dims=(0,), start_index_map=(0,))


def _lane_bcast(v16, lane):
    return lax.gather(v16, jnp.full((16, 1), lane, jnp.int32), _BCAST_DN, (1,),
                      mode=lax.GatherScatterMode.PROMISE_IN_BOUNDS)


@functools.lru_cache(maxsize=None)
def _make_fused(nnz_pad):
    per_tile = nnz_pad // NUM_TILES
    nbatch = per_tile // K
    mesh = plsc.VectorSubcoreMesh(core_axis_name="c", subcore_axis_name="s")

    out_types = (
        jax.ShapeDtypeStruct((N, B), jnp.float32),
        jax.ShapeDtypeStruct((NUM_CORES, MP, B), jnp.float32),
        jax.ShapeDtypeStruct((NUM_CORES, N, B), jnp.float32),
        jax.ShapeDtypeStruct((N, B), jnp.float32),
        jax.ShapeDtypeStruct((N, B), jnp.float32),
    )

    @functools.partial(
        pl.kernel,
        out_type=out_types,
        mesh=mesh,
        compiler_params=pltpu.CompilerParams(use_tc_tiling_on_sc=False),
        scratch_types=[
            pltpu.VMEM((RING, 2, K), jnp.int32),
            pltpu.VMEM((per_tile,), jnp.float32),
            pltpu.VMEM((K, B), jnp.float32),
            pltpu.VMEM((K, B), jnp.float32),
            pltpu.VMEM((K, B), jnp.float32),
            pltpu.VMEM((K, B), jnp.float32),
            pltpu.VMEM((K, B), jnp.float32),
            pltpu.VMEM((K, B), jnp.float32),
            pltpu.VMEM_SHARED((MP, B), jnp.float32),
            pltpu.VMEM_SHARED((MP, B), jnp.float32),
            pltpu.SemaphoreType.DMA,
            pltpu.SemaphoreType.DMA,
            pltpu.SemaphoreType.DMA,
            pltpu.SemaphoreType.DMA,
            pltpu.SemaphoreType.DMA,
            pltpu.SemaphoreType.DMA,
            pltpu.SemaphoreType.REGULAR,
        ],
    )
    def fused(v_hbm, idx2_hbm, val_hbm,
              res_hbm, up_hbm, qp_hbm, tcur_hbm, tprev_hbm,
              ring, val_v, rows_a, rows_b, d0, d1, d2, zb, acc,
              src_sp, gsa, gsb, ssa, ssb, stg, psem, bsem):
        cid = lax.axis_index("c")
        sid = lax.axis_index("s")
        wid = cid * NUM_SUBCORES + sid
        ebase = wid * per_tile
        rbase = wid * nbatch

        def gbar():
            plsc.subcore_barrier()
            @pl.when(sid == 0)
            def _():
                pl.semaphore_signal(bsem, 1, core_index=1 - cid)
                pl.semaphore_wait(bsem, 1)
            plsc.subcore_barrier()

        pltpu.async_copy(val_hbm.at[pl.ds(ebase, per_tile)], val_v, stg)

        zero16 = jnp.zeros((16,), jnp.float32)
        def zrow(i, carry):
            zb[i, pl.ds(0, 16)] = zero16
            zb[i, pl.ds(16, 16)] = zero16
            return carry
        lax.fori_loop(0, K, zrow, 0)

        def zero_acc_issue(rpt, sem):
            full, rem = divmod(rpt, K)
            for z in range(full):
                pltpu.async_copy(zb, acc.at[pl.ds(sid * rpt + z * K, K)], sem)
            if rem:
                pltpu.async_copy(zb.at[pl.ds(0, rem)],
                                 acc.at[pl.ds(sid * rpt + full * K, rem)], sem)

        def zero_acc_wait(rpt, sem):
            full, rem = divmod(rpt, K)
            for z in range(full):
                pltpu.make_async_copy(
                    zb, acc.at[pl.ds(sid * rpt + z * K, K)], sem).wait()
            if rem:
                pltpu.make_async_copy(
                    zb.at[pl.ds(0, rem)],
                    acc.at[pl.ds(sid * rpt + full * K, rem)], sem).wait()

        def scale(buf, i):
            @plsc.parallel_loop(0, K // 16)
            def _(g):
                val16 = val_v[pl.ds(i * K + g * 16, 16)]
                for kk in range(16):
                    e = g * 16 + kk
                    bv = _lane_bcast(val16, kk)
                    buf[e, pl.ds(0, 16)] = buf[e, pl.ds(0, 16)] * bv
                    buf[e, pl.ds(16, 16)] = buf[e, pl.ds(16, 16)] * bv

        def stage_rows(src_hbm, nrows):
            per = nrows // NUM_SUBCORES
            start = sid * per
            full, rem = divmod(per, K)
            for z in range(full):
                pltpu.async_copy(src_hbm.at[pl.ds(start + z * K, K)],
                                 src_sp.at[pl.ds(start + z * K, K)], stg)
            if rem:
                pltpu.async_copy(src_hbm.at[pl.ds(start + full * K, rem)],
                                 src_sp.at[pl.ds(start + full * K, rem)], stg)
            for z in range(full):
                pltpu.make_async_copy(src_hbm.at[pl.ds(start + z * K, K)],
                                      src_sp.at[pl.ds(start + z * K, K)], stg).wait()
            if rem:
                pltpu.make_async_copy(src_hbm.at[pl.ds(start + full * K, rem)],
                                      src_sp.at[pl.ds(start + full * K, rem)], stg).wait()

        def prefetch_idx(j):
            pltpu.async_copy(idx2_hbm.at[rbase + j], ring.at[j % RING], psem)

        def wait_idx():
            pltpu.make_async_copy(idx2_hbm.at[0], ring.at[0], psem).wait()

        def run_spmm(gsel, ssel):
            def issue_gather(i, buf, sem):
                pltpu.async_copy(src_sp.at[ring.at[i % RING, gsel]], buf, sem)

            def issue_scatter(i, buf, sem):
                pltpu.async_copy(buf, acc.at[ring.at[i % RING, ssel]], sem,
                                 add=True)

            def wait_rows_sem(buf, sem):
                pltpu.make_async_copy(src_sp.at[pl.ds(0, K)], buf, sem).wait()

            for j in range(min(RING, nbatch)):
                prefetch_idx(j)
            wait_idx()
            wait_idx()
            issue_gather(0, rows_a, gsa)
            issue_gather(1, rows_b, gsb)
            wait_rows_sem(rows_a, gsa)
            scale(rows_a, 0)
            issue_scatter(0, rows_a, ssa)

            def body(i, carry):
                @pl.when(i % 2 == 1)
                def _():
                    wait_rows_sem(rows_a, ssa)
                    @pl.when(i + RING - 1 < nbatch)
                    def _():
                        prefetch_idx(i + RING - 1)
                    @pl.when(i + 1 < nbatch)
                    def _():
                        wait_idx()
                        issue_gather(i + 1, rows_a, gsa)
                    wait_rows_sem(rows_b, gsb)
                    scale(rows_b, i)
                    issue_scatter(i, rows_b, ssb)

                @pl.when(i % 2 == 0)
                def _():
                    wait_rows_sem(rows_b, ssb)
                    @pl.when(i + RING - 1 < nbatch)
                    def _():
                        prefetch_idx(i + RING - 1)
                    @pl.when(i + 1 < nbatch)
                    def _():
                        wait_idx()
                        issue_gather(i + 1, rows_b, gsb)
                    wait_rows_sem(rows_a, gsa)
                    scale(rows_a, i)
                    issue_scatter(i, rows_a, ssa)

                return carry

            lax.fori_loop(1, nbatch, body, 0)
            wait_rows_sem(rows_b, ssb)

        def copy_acc_out(rpt, dst_hbm):
            full, rem = divmod(rpt, K)
            for z in range(full):
                pltpu.async_copy(acc.at[pl.ds(sid * rpt + z * K, K)],
                                 dst_hbm.at[cid, pl.ds(sid * rpt + z * K, K)], gsa)
            if rem:
                pltpu.async_copy(acc.at[pl.ds(sid * rpt + full * K, rem)],
                                 dst_hbm.at[cid, pl.ds(sid * rpt + full * K, rem)], gsa)
            for z in range(full):
                pltpu.make_async_copy(
                    acc.at[pl.ds(sid * rpt + z * K, K)],
                    dst_hbm.at[cid, pl.ds(sid * rpt + z * K, K)], gsa).wait()
            if rem:
                pltpu.make_async_copy(
                    acc.at[pl.ds(sid * rpt + full * K, rem)],
                    dst_hbm.at[cid, pl.ds(sid * rpt + full * K, rem)], gsa).wait()

        def add_rows(dst, a, b, n):
            def arow(r, carry):
                dst[r, pl.ds(0, 16)] = a[r, pl.ds(0, 16)] + b[r, pl.ds(0, 16)]
                dst[r, pl.ds(16, 16)] = a[r, pl.ds(16, 16)] + b[r, pl.ds(16, 16)]
                return carry
            lax.fori_loop(0, n, arow, 0)

        def dense_combine():
            base = sid * RPT_U
            full, rem = divmod(RPT_U, K)
            nch = full + (1 if rem else 0)
            slots = ((rows_a, rows_b, gsa), (d0, d1, gsb))

            def cnof(z):
                return K if z < full else rem

            def load(z):
                ba, bb, sem = slots[z % 2]
                cn = cnof(z)
                off = base + z * K
                pltpu.async_copy(up_hbm.at[0, pl.ds(off, cn)], ba.at[pl.ds(0, cn)], sem)
                pltpu.async_copy(up_hbm.at[1, pl.ds(off, cn)], bb.at[pl.ds(0, cn)], sem)

            def wait_load(z):
                ba, bb, sem = slots[z % 2]
                cn = cnof(z)
                off = base + z * K
                pltpu.make_async_copy(up_hbm.at[0, pl.ds(off, cn)], ba.at[pl.ds(0, cn)], sem).wait()
                pltpu.make_async_copy(up_hbm.at[1, pl.ds(off, cn)], bb.at[pl.ds(0, cn)], sem).wait()

            def wait_write(z):
                ba = slots[z % 2][0]
                cn = cnof(z)
                off = base + z * K
                pltpu.make_async_copy(ba.at[pl.ds(0, cn)], src_sp.at[pl.ds(off, cn)], ssb).wait()

            load(0)
            for z in range(nch):
                if z + 1 < nch:
                    if z >= 1:
                        wait_write(z - 1)
                    load(z + 1)
                wait_load(z)
                ba, bb, _ = slots[z % 2]
                cn = cnof(z)
                add_rows(ba, ba, bb, cn)
                pltpu.async_copy(ba.at[pl.ds(0, cn)], src_sp.at[pl.ds(base + z * K, cn)], ssb)
            wait_write(nch - 2)
            wait_write(nch - 1)

        lane_ids = lax.iota(jnp.int32, 16)
        coeffs16 = jnp.zeros((16,), jnp.float32)
        for ci, cval in enumerate(_COEFFS):
            coeffs16 = jnp.where(lane_ids == ci, jnp.float32(cval), coeffs16)

        def dense_recurrence_first():
            base = wid * DSL_N
            for z in range(DSL_N // K):
                off = base + z * K
                pltpu.async_copy(qp_hbm.at[0, pl.ds(off, K)], rows_a, gsa)
                pltpu.async_copy(qp_hbm.at[1, pl.ds(off, K)], rows_b, gsb)
                pltpu.async_copy(v_hbm.at[pl.ds(off, K)], d0, ssa)
                pltpu.make_async_copy(qp_hbm.at[0, pl.ds(off, K)], rows_a, gsa).wait()
                pltpu.make_async_copy(qp_hbm.at[1, pl.ds(off, K)], rows_b, gsb).wait()
                pltpu.make_async_copy(v_hbm.at[pl.ds(off, K)], d0, ssa).wait()
                c0, c1 = _COEFFS[0], _COEFFS[1]
                ch, cm = 1.0 / _HALF, _MID / _HALF
                def row1(r, carry):
                    for h in (0, 16):
                        a = rows_a[r, pl.ds(h, 16)] + rows_b[r, pl.ds(h, 16)]
                        vv = d0[r, pl.ds(h, 16)]
                        t1 = a * ch - vv * cm
                        rows_a[r, pl.ds(h, 16)] = t1
                        rows_b[r, pl.ds(h, 16)] = vv * c0 + t1 * c1
                    return carry
                lax.fori_loop(0, K, row1, 0)
                pltpu.sync_copy(d0, tprev_hbm.at[pl.ds(off, K)])
                pltpu.sync_copy(rows_a, tcur_hbm.at[pl.ds(off, K)])
                pltpu.sync_copy(rows_b, res_hbm.at[pl.ds(off, K)])

        def dense_recurrence_step(k):
            ckv = _lane_bcast(coeffs16, k)
            base = wid * DSL_N
            for z in range(DSL_N // K):
                off = base + z * K
                pltpu.async_copy(qp_hbm.at[0, pl.ds(off, K)], rows_a, gsa)
                pltpu.async_copy(qp_hbm.at[1, pl.ds(off, K)], rows_b, gsb)
                pltpu.async_copy(tcur_hbm.at[pl.ds(off, K)], d0, ssa)
                pltpu.async_copy(tprev_hbm.at[pl.ds(off, K)], d1, ssb)
                pltpu.async_copy(res_hbm.at[pl.ds(off, K)], d2, stg)
                pltpu.make_async_copy(qp_hbm.at[0, pl.ds(off, K)], rows_a, gsa).wait()
                pltpu.make_async_copy(qp_hbm.at[1, pl.ds(off, K)], rows_b, gsb).wait()
                pltpu.make_async_copy(tcur_hbm.at[pl.ds(off, K)], d0, ssa).wait()
                pltpu.make_async_copy(tprev_hbm.at[pl.ds(off, K)], d1, ssb).wait()
                pltpu.make_async_copy(res_hbm.at[pl.ds(off, K)], d2, stg).wait()
                c2h, c2m = 2.0 / _HALF, 2.0 * _MID / _HALF
                def rowk(r, carry):
                    for h in (0, 16):
                        a = rows_a[r, pl.ds(h, 16)] + rows_b[r, pl.ds(h, 16)]
                        tn = a * c2h - d0[r, pl.ds(h, 16)] * c2m - d1[r, pl.ds(h, 16)]
                        rows_a[r, pl.ds(h, 16)] = tn
                        rows_b[r, pl.ds(h, 16)] = d2[r, pl.ds(h, 16)] + tn * ckv
                    return carry
                lax.fori_loop(0, K, rowk, 0)
                pltpu.sync_copy(d0, tprev_hbm.at[pl.ds(off, K)])
                pltpu.sync_copy(rows_a, tcur_hbm.at[pl.ds(off, K)])
                pltpu.sync_copy(rows_b, res_hbm.at[pl.ds(off, K)])

        pltpu.make_async_copy(val_hbm.at[pl.ds(ebase, per_tile)], val_v, stg).wait()

        def iteration(k, src_hbm, dense_phase):
            zero_acc_issue(RPT_U, ssa)
            stage_rows(src_hbm, N)
            zero_acc_wait(RPT_U, ssa)
            plsc.subcore_barrier()
            run_spmm(0, 1)
            plsc.subcore_barrier()
            copy_acc_out(RPT_U, up_hbm)
            gbar()
            zero_acc_issue(RPT_N, stg)
            dense_combine()
            zero_acc_wait(RPT_N, stg)
            plsc.subcore_barrier()
            run_spmm(1, 0)
            plsc.subcore_barrier()
            copy_acc_out(RPT_N, qp_hbm)
            gbar()
            dense_phase(k)
            gbar()

        iteration(1, v_hbm, lambda k: dense_recurrence_first())

        def body(k, carry):
            iteration(k, tcur_hbm, dense_recurrence_step)
            return carry
        lax.fori_loop(2, DEGREE + 1, body, 0)

    return fused


def kernel(rating_matrix, x_row, x_col, x_val):
    nnz = x_row.shape[0]
    nnz_pad = -(-nnz // (NUM_TILES * K * 2)) * (NUM_TILES * K * 2)
    pad = nnz_pad - nnz

    row = jnp.concatenate([x_row.astype(jnp.int32), jnp.zeros((pad,), jnp.int32)])
    col = jnp.concatenate([x_col.astype(jnp.int32), jnp.zeros((pad,), jnp.int32)])
    val = jnp.concatenate([x_val.astype(jnp.float32), jnp.zeros((pad,), jnp.float32)])
    idx2 = jnp.stack([col.reshape(-1, K), row.reshape(-1, K)], axis=1)

    fused = _make_fused(nnz_pad)
    v = rating_matrix.T.reshape(N, B)
    res = fused(v, idx2, val)[0]
    return res.T
